# baseline (speedup 1.0000x reference)
import functools

import jax
import jax.numpy as jnp
from jax import lax
from jax.experimental import pallas as pl
from jax.experimental.pallas import tpu as pltpu
from jax.experimental.pallas import tpu_sc as plsc

_N = 128
_K_OTHER = 7
_K_TOT = 8
_B = 256
_ROWS = _B * _N
_NW = 32
_RPW = _ROWS // _NW
_MC = 64
_G = 16
_NP = _N + 1
_NSTREAM = 4

_NEG_INF = float("-inf")


def _sc_body(x_hbm, o_hbm, xin0, xin1, xbuf, ebuf, cbuf0, cbuf1, eyebuf,
             sem0, sem1, insem):
    nc = 2
    wid = lax.axis_index("s") * nc + lax.axis_index("c")
    wbase = wid * _RPW
    iota = lax.iota(jnp.int32, 16)

    def eye_init(i, _):
        for c in range(_N // 16):
            eyebuf[i, pl.ds(c * 16, 16)] = jnp.where(
                iota == i - c * 16, 1.0, 0.0
            ).astype(jnp.float32)
        return 0

    lax.fori_loop(0, _N, eye_init, 0)

    xins = (xin0, xin1)
    cbufs = (cbuf0, cbuf1)
    sems = (sem0, sem1)
    nchunks = _RPW // _MC

    def in_wait(xin):
        pltpu.make_async_copy(
            x_hbm.at[pl.ds(wbase, _MC)], xin.at[:, pl.ds(0, _N)], insem
        ).wait()

    def out_issue(cbuf, gb0, abase, sem):
        pltpu.async_copy(eyebuf.at[pl.ds(abase, _MC)],
                         o_hbm.at[pl.ds(gb0, _MC), 0], sem)
        for k in range(1, _K_TOT):
            pltpu.async_copy(
                cbuf.at[:, pl.ds(0, _N)], o_hbm.at[pl.ds(gb0, _MC), k], sem
            )

    def out_wait(cbuf, gb0, abase, sem):
        pltpu.make_async_copy(eyebuf.at[pl.ds(abase, _MC)],
                              o_hbm.at[pl.ds(gb0, _MC), 0], sem).wait()
        for k in range(1, _K_TOT):
            pltpu.make_async_copy(
                cbuf.at[:, pl.ds(0, _N)], o_hbm.at[pl.ds(gb0, _MC), k], sem
            ).wait()

    pltpu.async_copy(x_hbm.at[pl.ds(wbase, _MC)], xin0.at[:, pl.ds(0, _N)],
                     insem)

    for mc in range(2):
        xin, cbuf, sem = xins[mc], cbufs[mc], sems[mc]
        gb0 = wbase + mc * _MC
        abase = mc * _MC
        in_wait(xin)
        pltpu.async_copy(
            x_hbm.at[pl.ds(wbase + (mc + 1) * _MC, _MC)],
            xins[1 - mc].at[:, pl.ds(0, _N)], insem
        )
        _compute_chunk(xin, cbuf, xbuf, ebuf, iota, None, abase)
        out_issue(cbuf, gb0, abase, sem)

    def macro_pair(cc, _):
        for sub in range(2):
            mc = 2 * cc + sub
            gb0 = wbase + mc * _MC
            abase = sub * _MC
            xin, cbuf, sem = xins[sub], cbufs[sub], sems[sub]
            in_wait(xin)
            nxt = jnp.minimum(mc + 1, nchunks - 1)
            pltpu.async_copy(
                x_hbm.at[pl.ds(wbase + nxt * _MC, _MC)],
                xins[1 - sub].at[:, pl.ds(0, _N)], insem
            )
            drain = functools.partial(out_wait, cbuf, gb0 - 2 * _MC, abase,
                                      sem)
            _compute_chunk(xin, cbuf, xbuf, ebuf, iota, drain, abase)
            out_issue(cbuf, gb0, abase, sem)
        return 0

    lax.fori_loop(1, nchunks // 2, macro_pair, 0)

    for sub in range(2):
        mc = nchunks - 2 + sub
        out_wait(cbufs[sub], wbase + mc * _MC, sub * _MC, sems[sub])
    pltpu.make_async_copy(
        x_hbm.at[pl.ds(wbase, _MC)], xin0.at[:, pl.ds(0, _N)], insem
    ).wait()


def _compute_chunk(xin, cbuf, xbuf, ebuf, iota, drain, abase):
    for g in range(_MC // _G):
        rb = g * _G
        ibase = abase + rb
        rows = rb + iota

        span = _N // _NSTREAM

        ninf = jnp.full((16,), _NEG_INF)
        xs = plsc.load_gather(xin, [rows, ibase + iota])
        plsc.store_scatter(xin, [rows, ibase + iota], ninf)
        tinit = tuple(
            tuple(ninf for _ in range(_K_OTHER)) for _ in range(_NSTREAM)
        )

        @plsc.parallel_loop(0, span, unroll=4, carry=tinit)
        def tstr(i, carry):
            ts = [list(carry[s]) for s in range(_NSTREAM)]
            for s in range(_NSTREAM):
                j = i + s * span
                jv = jnp.full((16,), j, dtype=jnp.int32)
                v = plsc.load_gather(xin, [rows, jv])
                xbuf[j, :] = v
                t = ts[s]
                for k in range(_K_OTHER):
                    hi = jnp.maximum(t[k], v)
                    v = jnp.minimum(t[k], v)
                    t[k] = hi
            return tuple(tuple(t) for t in ts)

        t = list(tstr[0])
        for s in range(1, _NSTREAM):
            for k in range(_K_OTHER):
                v = tstr[s][k]
                for q in range(_K_OTHER):
                    hi = jnp.maximum(t[q], v)
                    v = jnp.minimum(t[q], v)
                    t[q] = hi

        c0 = t[0]
        thr = t[_K_OTHER - 1]
        nbig = jnp.zeros((16,), jnp.float32)
        for k in range(_K_OTHER - 1):
            nbig = nbig + jnp.where(t[k] > thr, 1.0, 0.0)
        need = 7.0 - nbig

        zeros = jnp.zeros((16,), jnp.float32)

        @plsc.parallel_loop(0, _N, unroll=8, carry=(zeros, zeros, zeros))
        def bcarry(j, carry):
            z, eqcnt, ssum = carry
            v = xbuf[j, :]
            e = jnp.exp(v - c0)
            z = z + e
            gt = v > thr
            eq = v == thr
            sel = gt | (eq & (eqcnt < need))
            eqcnt = eqcnt + jnp.where(eq, 1.0, 0.0)
            se = jnp.where(sel, e, 0.0)
            ssum = ssum + se
            ebuf[j, :] = se
            return z, eqcnt, ssum

        z, _, ssum = bcarry

        z = z + jnp.exp(xs - c0)
        inv = 1.0 / (ssum + 1e-8 * z)

        if g == 0 and drain is not None:
            drain()

        @plsc.parallel_loop(0, _N, unroll=8)
        def _(j):
            se = ebuf[j, :]
            w = se * inv
            jv = jnp.full((16,), j, dtype=jnp.int32)
            plsc.store_scatter(cbuf, [rows, jv], w)


@jax.jit
def kernel(scores):
    batch = scores.shape[0]
    x = scores.reshape(_ROWS, _N)
    mesh = plsc.VectorSubcoreMesh(core_axis_name="c", subcore_axis_name="s")
    run = pl.kernel(
        _sc_body,
        out_type=jax.ShapeDtypeStruct((_ROWS, _K_TOT, _N), jnp.float32),
        mesh=mesh,
        compiler_params=pltpu.CompilerParams(needs_layout_passes=False),
        scratch_types=[
            pltpu.VMEM((_MC, _NP), jnp.float32),
            pltpu.VMEM((_MC, _NP), jnp.float32),
            pltpu.VMEM((_N, 16), jnp.float32),
            pltpu.VMEM((_N, 16), jnp.float32),
            pltpu.VMEM((_MC, _NP), jnp.float32),
            pltpu.VMEM((_MC, _NP), jnp.float32),
            pltpu.VMEM((_N, _N), jnp.float32),
            pltpu.SemaphoreType.DMA,
            pltpu.SemaphoreType.DMA,
            pltpu.SemaphoreType.DMA,
        ],
    )
    out = run(x)
    return out.reshape(batch, _N, _K_TOT, _N)


# speedup vs baseline: 1.0255x; 1.0255x over previous
import functools

import jax
import jax.numpy as jnp
from jax import lax
from jax.experimental import pallas as pl
from jax.experimental.pallas import tpu as pltpu
from jax.experimental.pallas import tpu_sc as plsc

_N = 128
_K_OTHER = 7
_K_TOT = 8
_B = 256
_ROWS = _B * _N
_NW = 32
_RPW = _ROWS // _NW
_MC = 128
_G = 16
_NP = _N + 1
_NSTREAM = 4

_NEG_INF = float("-inf")


def _sc_body(x_hbm, o_hbm, xin0, xin1, xbuf, ebuf, sbuf0, sbuf1, eyebuf, sem,
             insem):
    nc = 2
    wid = lax.axis_index("s") * nc + lax.axis_index("c")
    wbase = wid * _RPW
    iota = lax.iota(jnp.int32, 16)

    def eye_init(i, _):
        for c in range(_N // 16):
            eyebuf[i, pl.ds(c * 16, 16)] = jnp.where(
                iota == i - c * 16, 1.0, 0.0
            ).astype(jnp.float32)
        return 0

    lax.fori_loop(0, _N, eye_init, 0)

    sbufs = (sbuf0, sbuf1)
    xins = (xin0, xin1)

    pltpu.async_copy(x_hbm.at[pl.ds(wbase, _MC)], xin0.at[:, pl.ds(0, _N)], insem)

    def macro_pair(cc, _):
        for sub in range(2):
            mc = 2 * cc + sub
            xin = xins[sub]
            xin_next = xins[1 - sub]
            gb0 = wbase + mc * _MC
            pltpu.make_async_copy(
                x_hbm.at[pl.ds(wbase, _MC)], xin.at[:, pl.ds(0, _N)], insem
            ).wait()
            nxt = jnp.minimum(mc + 1, _RPW // _MC - 1)
            pltpu.async_copy(
                x_hbm.at[pl.ds(wbase + nxt * _MC, _MC)], xin_next.at[:, pl.ds(0, _N)], insem
            )
            _process_chunk(o_hbm, xin, xbuf, ebuf, sbufs, eyebuf, sem, gb0,
                           iota)
        return 0

    lax.fori_loop(0, _RPW // _MC // 2, macro_pair, 0)
    pltpu.make_async_copy(x_hbm.at[pl.ds(wbase, _MC)], xin0.at[:, pl.ds(0, _N)], insem).wait()


def _process_chunk(o_hbm, xin, xbuf, ebuf, sbufs, eyebuf, sem, gb0, iota):
    if True:
        pending = [[], []]
        for g in range(_MC // _G):
            rb = g * _G
            ibase = rb
            rows = rb + iota
            par = g % 2
            sbuf = sbufs[par]

            span = _N // _NSTREAM

            ninf = jnp.full((16,), _NEG_INF)
            xs = plsc.load_gather(xin, [rows, ibase + iota])
            plsc.store_scatter(xin, [rows, ibase + iota], ninf)
            tinit = tuple(
                tuple(ninf for _ in range(_K_OTHER)) for _ in range(_NSTREAM)
            )

            @plsc.parallel_loop(0, span, unroll=4, carry=tinit)
            def tstr(i, carry):
                ts = [list(carry[s]) for s in range(_NSTREAM)]
                for s in range(_NSTREAM):
                    j = i + s * span
                    jv = jnp.full((16,), j, dtype=jnp.int32)
                    v = plsc.load_gather(xin, [rows, jv])
                    xbuf[j, :] = v
                    t = ts[s]
                    for k in range(_K_OTHER):
                        hi = jnp.maximum(t[k], v)
                        v = jnp.minimum(t[k], v)
                        t[k] = hi
                return tuple(tuple(t) for t in ts)

            t = list(tstr[0])
            for s in range(1, _NSTREAM):
                for k in range(_K_OTHER):
                    v = tstr[s][k]
                    for q in range(_K_OTHER):
                        hi = jnp.maximum(t[q], v)
                        v = jnp.minimum(t[q], v)
                        t[q] = hi

            c0 = t[0]
            thr = t[_K_OTHER - 1]
            nbig = jnp.zeros((16,), jnp.float32)
            for k in range(_K_OTHER - 1):
                nbig = nbig + jnp.where(t[k] > thr, 1.0, 0.0)
            need = 7.0 - nbig

            zeros = jnp.zeros((16,), jnp.float32)

            @plsc.parallel_loop(0, _N, unroll=8, carry=(zeros, zeros, zeros))
            def bcarry(j, carry):
                z, eqcnt, ssum = carry
                v = xbuf[j, :]
                e = jnp.exp(v - c0)
                z = z + e
                gt = v > thr
                eq = v == thr
                sel = gt | (eq & (eqcnt < need))
                eqcnt = eqcnt + jnp.where(eq, 1.0, 0.0)
                se = jnp.where(sel, e, 0.0)
                ssum = ssum + se
                ebuf[j, :] = se
                return z, eqcnt, ssum

            z, _, ssum = bcarry

            z = z + jnp.exp(xs - c0)
            inv = 1.0 / (ssum + 1e-8 * z)

            for cp in pending[par]:
                cp.wait()
            pending[par] = []

            @plsc.parallel_loop(0, _N, unroll=8)
            def _(j):
                se = ebuf[j, :]
                w = se * inv
                jv = jnp.full((16,), j, dtype=jnp.int32)
                plsc.store_scatter(sbuf, [iota, jv], w)

            orow = (gb0 + rb + iota) * _K_TOT
            pending[par].append(
                pltpu.async_copy(eyebuf.at[pl.ds(ibase, _G)], o_hbm.at[orow], sem)
            )
            for k in range(1, _K_TOT):
                pending[par].append(
                    pltpu.async_copy(sbuf.at[:, pl.ds(0, _N)], o_hbm.at[orow + k], sem)
                )

        for plist in pending:
            for cp in plist:
                cp.wait()


@jax.jit
def kernel(scores):
    batch = scores.shape[0]
    x = scores.reshape(_ROWS, _N)
    mesh = plsc.VectorSubcoreMesh(core_axis_name="c", subcore_axis_name="s")
    run = pl.kernel(
        _sc_body,
        out_type=jax.ShapeDtypeStruct((_ROWS * _K_TOT, _N), jnp.float32),
        mesh=mesh,
        compiler_params=pltpu.CompilerParams(needs_layout_passes=False),
        scratch_types=[
            pltpu.VMEM((_MC, _NP), jnp.float32),
            pltpu.VMEM((_MC, _NP), jnp.float32),
            pltpu.VMEM((_N, 16), jnp.float32),
            pltpu.VMEM((_N, 16), jnp.float32),
            pltpu.VMEM((_G, _NP), jnp.float32),
            pltpu.VMEM((_G, _NP), jnp.float32),
            pltpu.VMEM((_N, _N), jnp.float32),
            pltpu.SemaphoreType.DMA,
            pltpu.SemaphoreType.DMA,
        ],
    )
    out = run(x)
    return out.reshape(batch, _N, _K_TOT, _N)
